# Initial kernel scaffold; baseline (speedup 1.0000x reference)
import jax
import jax.numpy as jnp
from jax.experimental import pallas as pl


def kernel(x, edge_index, batch, W_in, b_in, eps, W1, b1, g1, be1, W2, b2, g2, be2, Wh1, bh1, Wh2, bh2):
    raise NotImplementedError("write your pallas kernel here")


# speedup vs baseline: 4.4336x; 4.4336x over previous
import functools

import jax
import jax.numpy as jnp
from jax import lax
from jax.experimental import pallas as pl
from jax.experimental.pallas import tpu as pltpu
from jax.experimental.pallas import tpu_sc as plsc

N = 10000
E = 320000
D = 128
L = 5
G = 256

_NC = 2
_NS = 16
_NW = _NC * _NS
_EW = E // _NW
_C = 80
_NCH = _EW // _C
_RPS = 624
_TAIL = N - _NS * _RPS

_PREC = jax.lax.Precision.DEFAULT
_PREC_HI = jax.lax.Precision.HIGHEST


def _make_agg():
    mesh = plsc.VectorSubcoreMesh(core_axis_name="c", subcore_axis_name="s")

    @functools.partial(
        pl.kernel,
        mesh=mesh,
        out_type=jax.ShapeDtypeStruct((_NC * N, D), jnp.float32),
        scratch_types=[
            pltpu.VMEM((_C,), jnp.int32),
            pltpu.VMEM((_C,), jnp.int32),
            pltpu.VMEM((_C, D), jnp.float32),
            pltpu.VMEM_SHARED((N, D), jnp.float32),
            pltpu.SemaphoreType.DMA,
        ],
    )
    def agg(h_hbm, src_hbm, dst_hbm, zero_hbm, out_hbm,
            src_v, dst_v, rows_v, acc_sh, sem):
        cid = lax.axis_index("c")
        sid = lax.axis_index("s")
        pltpu.sync_copy(zero_hbm.at[pl.ds(sid * _RPS, _RPS)],
                        acc_sh.at[pl.ds(sid * _RPS, _RPS)])

        @pl.when(sid == 0)
        def _zero_tail():
            pltpu.sync_copy(zero_hbm.at[pl.ds(_NS * _RPS, _TAIL)],
                            acc_sh.at[pl.ds(_NS * _RPS, _TAIL)])

        plsc.subcore_barrier()

        base = (cid * _NS + sid) * _EW

        def body(i, carry):
            off = base + i * _C
            pltpu.sync_copy(src_hbm.at[pl.ds(off, _C)], src_v)
            pltpu.sync_copy(dst_hbm.at[pl.ds(off, _C)], dst_v)
            pltpu.async_copy(h_hbm.at[src_v], rows_v, sem).wait()
            pltpu.sync_copy(rows_v, acc_sh.at[dst_v], add=True)
            return carry

        lax.fori_loop(0, _NCH, body, 0)
        plsc.subcore_barrier()
        pltpu.sync_copy(acc_sh.at[pl.ds(sid * _RPS, _RPS)],
                        out_hbm.at[pl.ds(cid * N + sid * _RPS, _RPS)])

        @pl.when(sid == 0)
        def _out_tail():
            pltpu.sync_copy(acc_sh.at[pl.ds(_NS * _RPS, _TAIL)],
                            out_hbm.at[pl.ds(cid * N + _NS * _RPS, _TAIL)])

    return agg


def _in_body(x_ref, w_ref, b_ref, o_ref):
    o_ref[:] = jnp.maximum(
        jnp.dot(x_ref[:], w_ref[:], preferred_element_type=jnp.float32,
                precision=_PREC) + b_ref[:], 0.0)


def _layer_body(eps_ref, h_ref, p_ref, w1_ref, b1_ref, g1_ref, be1_ref,
                w2_ref, b2_ref, g2_ref, be2_ref, o_ref):
    z = (1.0 + eps_ref[0, 0]) * h_ref[:] + p_ref[:N] + p_ref[N:]
    u = jnp.dot(z, w1_ref[:], preferred_element_type=jnp.float32,
                precision=_PREC) + b1_ref[:]
    m = jnp.mean(u, axis=0, keepdims=True)
    c = u - m
    v = jnp.mean(c * c, axis=0, keepdims=True)
    u = jnp.maximum(g1_ref[:] * c / jnp.sqrt(v + 1e-5) + be1_ref[:], 0.0)
    w = jnp.dot(u, w2_ref[:], preferred_element_type=jnp.float32,
                precision=_PREC) + b2_ref[:]
    m2 = jnp.mean(w, axis=0, keepdims=True)
    c2 = w - m2
    v2 = jnp.mean(c2 * c2, axis=0, keepdims=True)
    o_ref[:] = jnp.maximum(g2_ref[:] * c2 / jnp.sqrt(v2 + 1e-5) + be2_ref[:],
                           0.0)


def _pool_body(h_ref, batch_ref, w1_ref, b1_ref, w2_ref, b2_ref, o_ref):
    ids = jax.lax.broadcasted_iota(jnp.int32, (G, N), 0)
    oh = (ids == batch_ref[:]).astype(jnp.float32)
    cnt = jnp.maximum(jnp.sum(oh, axis=1, keepdims=True), 1.0)
    pooled = jnp.dot(oh, h_ref[:], preferred_element_type=jnp.float32,
                     precision=_PREC_HI) / cnt
    hid = jnp.maximum(
        jnp.dot(pooled, w1_ref[:], preferred_element_type=jnp.float32,
                precision=_PREC) + b1_ref[:], 0.0)
    o_ref[:] = jnp.dot(hid, w2_ref[:], preferred_element_type=jnp.float32,
                       precision=_PREC) + b2_ref[:]


_in_tc = pl.pallas_call(_in_body,
                        out_shape=jax.ShapeDtypeStruct((N, D), jnp.float32))
_layer_tc = pl.pallas_call(_layer_body,
                           out_shape=jax.ShapeDtypeStruct((N, D), jnp.float32))
_pool_tc = pl.pallas_call(_pool_body,
                          out_shape=jax.ShapeDtypeStruct((G, 1), jnp.float32))


def kernel(x, edge_index, batch, W_in, b_in, eps, W1, b1, g1, be1,
           W2, b2, g2, be2, Wh1, bh1, Wh2, bh2):
    src = edge_index[0]
    dst = edge_index[1]
    zeros = jnp.zeros((N, D), jnp.float32)
    agg = _make_agg()

    h = _in_tc(x, W_in, b_in.reshape(1, D))
    for l in range(L):
        parts = agg(h, src, dst, zeros)
        h = _layer_tc(eps[l].reshape(1, 1), h, parts,
                      W1[l], b1[l].reshape(1, D), g1[l].reshape(1, D),
                      be1[l].reshape(1, D),
                      W2[l], b2[l].reshape(1, D), g2[l].reshape(1, D),
                      be2[l].reshape(1, D))
    return _pool_tc(h, batch.reshape(1, N), Wh1, bh1.reshape(1, D // 2),
                    Wh2, bh2.reshape(1, 1))


# speedup vs baseline: 12.4665x; 2.8119x over previous
import functools

import jax
import jax.numpy as jnp
from jax import lax
from jax.experimental import pallas as pl
from jax.experimental.pallas import tpu as pltpu
from jax.experimental.pallas import tpu_sc as plsc

N = 10000
E = 320000
D = 128
L = 5
G = 256

_NC = 2
_NS = 16
_NW = _NC * _NS
_EW = E // _NW
_C = 40
_NCH = _EW // _C
_RPS = 624
_TAIL = N - _NS * _RPS

_PREC = jax.lax.Precision.DEFAULT
_PREC_HI = jax.lax.Precision.HIGHEST


_NB = 5
_GRP = _NCH // _NB


def _make_agg():
    mesh = plsc.VectorSubcoreMesh(core_axis_name="c", subcore_axis_name="s")

    @functools.partial(
        pl.kernel,
        mesh=mesh,
        out_type=jax.ShapeDtypeStruct((_NC * N, D), jnp.float32),
        scratch_types=[
            pltpu.VMEM((_EW,), jnp.int32),
            pltpu.VMEM((_EW,), jnp.int32),
            pltpu.VMEM((_NB, _C, D), jnp.float32),
            pltpu.VMEM_SHARED((N, D), jnp.float32),
            pltpu.SemaphoreType.DMA,
            pltpu.SemaphoreType.DMA,
            pltpu.SemaphoreType.DMA,
            pltpu.SemaphoreType.DMA,
            pltpu.SemaphoreType.DMA,
        ],
    )
    def agg(h_hbm, src_hbm, dst_hbm, zero_hbm, out_hbm,
            src_v, dst_v, rows_v, acc_sh, sem0, sem1, sem2, sem3, sem4):
        sems = (sem0, sem1, sem2, sem3, sem4)
        cid = lax.axis_index("c")
        sid = lax.axis_index("s")
        wid = cid * _NS + sid

        pltpu.sync_copy(src_hbm.at[pl.ds(wid * _EW, _EW)], src_v)
        pltpu.sync_copy(dst_hbm.at[pl.ds(wid * _EW, _EW)], dst_v)

        pltpu.sync_copy(zero_hbm.at[pl.ds(sid * _RPS, _RPS)],
                        acc_sh.at[pl.ds(sid * _RPS, _RPS)])

        @pl.when(sid == 0)
        def _zero_tail():
            pltpu.sync_copy(zero_hbm.at[pl.ds(_NS * _RPS, _TAIL)],
                            acc_sh.at[pl.ds(_NS * _RPS, _TAIL)])

        for b in range(_NB):
            pltpu.async_copy(h_hbm.at[src_v.at[pl.ds(b * _C, _C)]],
                             rows_v.at[b], sems[b])

        plsc.subcore_barrier()

        def group(g, carry):
            for b in range(_NB):
                ch = g * _NB + b
                pltpu.make_async_copy(
                    h_hbm.at[src_v.at[pl.ds(ch * _C, _C)]], rows_v.at[b],
                    sems[b]).wait()
                pltpu.sync_copy(rows_v.at[b],
                                acc_sh.at[dst_v.at[pl.ds(ch * _C, _C)]],
                                add=True)

                @pl.when(g + 1 < _GRP)
                def _refill():
                    pltpu.async_copy(
                        h_hbm.at[src_v.at[pl.ds((ch + _NB) * _C, _C)]],
                        rows_v.at[b], sems[b])
            return carry

        lax.fori_loop(0, _GRP, group, 0)
        plsc.subcore_barrier()
        pltpu.sync_copy(acc_sh.at[pl.ds(sid * _RPS, _RPS)],
                        out_hbm.at[pl.ds(cid * N + sid * _RPS, _RPS)])

        @pl.when(sid == 0)
        def _out_tail():
            pltpu.sync_copy(acc_sh.at[pl.ds(_NS * _RPS, _TAIL)],
                            out_hbm.at[pl.ds(cid * N + _NS * _RPS, _TAIL)])

    return agg


def _in_body(x_ref, w_ref, b_ref, o_ref):
    o_ref[:] = jnp.maximum(
        jnp.dot(x_ref[:], w_ref[:], preferred_element_type=jnp.float32,
                precision=_PREC) + b_ref[:], 0.0)


def _layer_body(eps_ref, h_ref, p_ref, w1_ref, b1_ref, g1_ref, be1_ref,
                w2_ref, b2_ref, g2_ref, be2_ref, o_ref):
    z = (1.0 + eps_ref[0, 0]) * h_ref[:] + p_ref[:N] + p_ref[N:]
    u = jnp.dot(z, w1_ref[:], preferred_element_type=jnp.float32,
                precision=_PREC) + b1_ref[:]
    m = jnp.mean(u, axis=0, keepdims=True)
    c = u - m
    v = jnp.mean(c * c, axis=0, keepdims=True)
    u = jnp.maximum(g1_ref[:] * c / jnp.sqrt(v + 1e-5) + be1_ref[:], 0.0)
    w = jnp.dot(u, w2_ref[:], preferred_element_type=jnp.float32,
                precision=_PREC) + b2_ref[:]
    m2 = jnp.mean(w, axis=0, keepdims=True)
    c2 = w - m2
    v2 = jnp.mean(c2 * c2, axis=0, keepdims=True)
    o_ref[:] = jnp.maximum(g2_ref[:] * c2 / jnp.sqrt(v2 + 1e-5) + be2_ref[:],
                           0.0)


def _pool_body(h_ref, batch_ref, w1_ref, b1_ref, w2_ref, b2_ref, o_ref):
    ids = jax.lax.broadcasted_iota(jnp.int32, (G, N), 0)
    oh = (ids == batch_ref[:]).astype(jnp.float32)
    cnt = jnp.maximum(jnp.sum(oh, axis=1, keepdims=True), 1.0)
    pooled = jnp.dot(oh, h_ref[:], preferred_element_type=jnp.float32,
                     precision=_PREC_HI) / cnt
    hid = jnp.maximum(
        jnp.dot(pooled, w1_ref[:], preferred_element_type=jnp.float32,
                precision=_PREC) + b1_ref[:], 0.0)
    o_ref[:] = jnp.dot(hid, w2_ref[:], preferred_element_type=jnp.float32,
                       precision=_PREC) + b2_ref[:]


_in_tc = pl.pallas_call(_in_body,
                        out_shape=jax.ShapeDtypeStruct((N, D), jnp.float32))
_layer_tc = pl.pallas_call(_layer_body,
                           out_shape=jax.ShapeDtypeStruct((N, D), jnp.float32))
_pool_tc = pl.pallas_call(_pool_body,
                          out_shape=jax.ShapeDtypeStruct((G, 1), jnp.float32))


def kernel(x, edge_index, batch, W_in, b_in, eps, W1, b1, g1, be1,
           W2, b2, g2, be2, Wh1, bh1, Wh2, bh2):
    src = edge_index[0]
    dst = edge_index[1]
    zeros = jnp.zeros((N, D), jnp.float32)
    agg = _make_agg()

    h = _in_tc(x, W_in, b_in.reshape(1, D))
    for l in range(L):
        parts = agg(h, src, dst, zeros)
        h = _layer_tc(eps[l].reshape(1, 1), h, parts,
                      W1[l], b1[l].reshape(1, D), g1[l].reshape(1, D),
                      be1[l].reshape(1, D),
                      W2[l], b2[l].reshape(1, D), g2[l].reshape(1, D),
                      be2[l].reshape(1, D))
    return _pool_tc(h, batch.reshape(1, N), Wh1, bh1.reshape(1, D // 2),
                    Wh2, bh2.reshape(1, 1))
